# Initial kernel scaffold; baseline (speedup 1.0000x reference)
import jax
import jax.numpy as jnp
from jax.experimental import pallas as pl


def kernel(input, dim):
    raise NotImplementedError("write your pallas kernel here")


# speedup vs baseline: 1.5103x; 1.5103x over previous
import functools

import jax
import jax.numpy as jnp
from jax import lax
from jax.experimental import pallas as pl
from jax.experimental.pallas import tpu as pltpu
from jax.experimental.pallas import tpu_sc as plsc

NC = 2
NS = 16
NW = NC * NS
L = 16

ROW = 49
N_ROWS = 256 * 2048
ROWS_PER_W = N_ROWS // NW
CH = 512
NCHUNK = ROWS_PER_W // CH
GROUPS = CH // L


def _insert4(m1, m2, m3, m4, v):
    t = jnp.minimum(m1, v)
    m1 = jnp.maximum(m1, v)
    u = jnp.minimum(m2, t)
    m2 = jnp.maximum(m2, t)
    w = jnp.minimum(m3, u)
    m3 = jnp.maximum(m3, u)
    m4 = jnp.maximum(m4, w)
    return m1, m2, m3, m4


def _sc_body(x_hbm, out_hbm, in0, in1, ob, s0, s1):
    wid = lax.axis_index("s") * NC + lax.axis_index("c")
    wbase = wid * (ROWS_PER_W * ROW)
    obase = wid * ROWS_PER_W

    riota = lax.broadcasted_iota(jnp.int32, (L,), 0) * ROW
    neg_inf = jnp.full((L,), -jnp.inf, dtype=jnp.float32)

    bufs = (in0, in1)
    sems = (s0, s1)

    def start(c):
        return pltpu.async_copy(
            x_hbm.at[pl.ds(wbase + c * (CH * ROW), CH * ROW)],
            bufs[c % 2],
            sems[c % 2],
        )

    pending = start(0)
    for c in range(NCHUNK):
        nxt = start(c + 1) if c + 1 < NCHUNK else None
        pending.wait()
        buf = bufs[c % 2]

        def group(g, carry):
            base = g * (L * ROW)
            m1 = m2 = m3 = m4 = neg_inf
            for j in range(ROW):
                idx = riota + (base + j)
                v = plsc.load_gather(buf, [idx])
                m1, m2, m3, m4 = _insert4(m1, m2, m3, m4, v)
            ob[pl.ds(g * L, L)] = (m1 + m2 + m3 + m4) * 0.25
            return carry

        lax.fori_loop(0, GROUPS, group, 0, unroll=False)
        pltpu.sync_copy(ob, out_hbm.at[pl.ds(obase + c * CH, CH)])
        pending = nxt


@jax.jit
def _apool_sc(xf):
    mesh = plsc.VectorSubcoreMesh(
        core_axis_name="c", subcore_axis_name="s", num_cores=NC, num_subcores=NS
    )
    return pl.kernel(
        _sc_body,
        out_type=jax.ShapeDtypeStruct((N_ROWS,), jnp.float32),
        mesh=mesh,
        compiler_params=pltpu.CompilerParams(needs_layout_passes=False),
        scratch_types=[
            pltpu.VMEM((CH * ROW,), jnp.float32),
            pltpu.VMEM((CH * ROW,), jnp.float32),
            pltpu.VMEM((CH,), jnp.float32),
            pltpu.SemaphoreType.DMA,
            pltpu.SemaphoreType.DMA,
        ],
    )(xf)


def kernel(input, dim):
    xf = input.reshape(-1)
    out = _apool_sc(xf)
    return out.reshape(-1, 2048, 1, 1)


# speedup vs baseline: 17.5033x; 11.5896x over previous
import jax
import jax.numpy as jnp
from jax import lax
from jax.experimental import pallas as pl
from jax.experimental.pallas import tpu as pltpu
from jax.experimental.pallas import tpu_sc as plsc

NC = 2
NS = 16
NW = NC * NS
L = 16

NPLANE = 49
PSIZE = 256 * 2048
PER_W = PSIZE // NW
CH = 1024
NCHUNK = PER_W // CH
GROUPS = CH // L


def _insert4(m1, m2, m3, m4, v):
    t = jnp.minimum(m1, v)
    m1 = jnp.maximum(m1, v)
    u = jnp.minimum(m2, t)
    m2 = jnp.maximum(m2, t)
    w = jnp.minimum(m3, u)
    m3 = jnp.maximum(m3, u)
    m4 = jnp.maximum(m4, w)
    return m1, m2, m3, m4


def _sc_body(x_hbm, out_hbm, in0, in1, ob, si0, si1, so):
    wid = lax.axis_index("s") * NC + lax.axis_index("c")
    wbase = wid * PER_W

    bufs = (in0, in1)
    sems = (si0, si1)

    def start(c):
        base = wbase + c * CH
        buf, sem = bufs[c % 2], sems[c % 2]
        return [
            pltpu.async_copy(
                x_hbm.at[pl.ds(p * PSIZE + base, CH)],
                buf.at[pl.ds(p * CH, CH)],
                sem,
            )
            for p in range(NPLANE)
        ]

    pending = start(0)
    out_pending = []
    for c in range(NCHUNK):
        nxt = start(c + 1) if c + 1 < NCHUNK else None
        for h in pending:
            h.wait()
        for h in out_pending:
            h.wait()
        buf = bufs[c % 2]

        def group(g, carry):
            s = g * L
            m1 = buf[pl.ds(s, L)]
            m2 = buf[pl.ds(CH + s, L)]
            m3 = buf[pl.ds(2 * CH + s, L)]
            m4 = buf[pl.ds(3 * CH + s, L)]
            lo12, m1 = jnp.minimum(m1, m2), jnp.maximum(m1, m2)
            lo34, m3 = jnp.minimum(m3, m4), jnp.maximum(m3, m4)
            m1, hi = jnp.maximum(m1, m3), jnp.minimum(m1, m3)
            m4, lo = jnp.minimum(lo12, lo34), jnp.maximum(lo12, lo34)
            m2 = jnp.maximum(hi, lo)
            m3 = jnp.minimum(hi, lo)
            for p in range(4, NPLANE):
                m1, m2, m3, m4 = _insert4(
                    m1, m2, m3, m4, buf[pl.ds(p * CH + s, L)]
                )
            ob[pl.ds(s, L)] = (m1 + m2 + m3 + m4) * 0.25
            return carry

        lax.fori_loop(0, GROUPS, group, 0)

        tile = wid * NCHUNK + c
        b_hi = tile // 16
        c_hi = tile % 16
        out_pending = [
            pltpu.async_copy(
                ob.at[pl.ds(r * 128, 128)],
                out_hbm.at[pl.ds(b_hi * 16384 + r * 2048 + c_hi * 128, 128)],
                so,
            )
            for r in range(8)
        ]
        pending = nxt

    for h in out_pending:
        h.wait()


@jax.jit
def _apool_sc(xf):
    mesh = plsc.VectorSubcoreMesh(
        core_axis_name="c", subcore_axis_name="s", num_cores=NC, num_subcores=NS
    )
    return pl.kernel(
        _sc_body,
        out_type=jax.ShapeDtypeStruct((PSIZE,), jnp.float32),
        mesh=mesh,
        compiler_params=pltpu.CompilerParams(needs_layout_passes=False),
        scratch_types=[
            pltpu.VMEM((NPLANE * CH,), jnp.float32),
            pltpu.VMEM((NPLANE * CH,), jnp.float32),
            pltpu.VMEM((CH,), jnp.float32),
            pltpu.SemaphoreType.DMA,
            pltpu.SemaphoreType.DMA,
            pltpu.SemaphoreType.DMA,
        ],
    )(xf)


def kernel(input, dim):
    xt = jnp.transpose(input, (2, 3, 0, 1))
    x5 = xt.reshape(49, 32, 8, 16, 128)
    x5 = jnp.transpose(x5, (0, 1, 3, 2, 4))
    xf = x5.reshape(-1)
    out = _apool_sc(xf)
    return out.reshape(256, 2048, 1, 1)


# speedup vs baseline: 21.2206x; 1.2124x over previous
import jax
import jax.numpy as jnp
from jax import lax
from jax.experimental import pallas as pl
from jax.experimental.pallas import tpu as pltpu
from jax.experimental.pallas import tpu_sc as plsc

NC = 2
NS = 16
NW = NC * NS
L = 16

NPLANE = 49
PSIZE = 256 * 2048
PER_W = PSIZE // NW
CH = 1024
NCHUNK = PER_W // CH
GROUPS = CH // L


def _sort4(a, b, c, d):
    a, b = jnp.maximum(a, b), jnp.minimum(a, b)
    c, d = jnp.maximum(c, d), jnp.minimum(c, d)
    a, c = jnp.maximum(a, c), jnp.minimum(a, c)
    b, d = jnp.maximum(b, d), jnp.minimum(b, d)
    b, c = jnp.maximum(b, c), jnp.minimum(b, c)
    return a, b, c, d


def _merge44(A, B):
    a1, a2, a3, a4 = A
    b1, b2, b3, b4 = B
    c1 = jnp.maximum(a1, b1)
    q = jnp.minimum(a1, b1)
    r = jnp.maximum(a3, b3)
    c2 = jnp.maximum(q, r)
    c3 = jnp.minimum(q, r)
    d1 = jnp.maximum(a2, b2)
    q2 = jnp.minimum(a2, b2)
    r2 = jnp.maximum(a4, b4)
    d2 = jnp.maximum(q2, r2)
    return c1, jnp.maximum(d1, c2), jnp.minimum(d1, c2), jnp.maximum(d2, c3)


def _insert4(m, v):
    m1, m2, m3, m4 = m
    t = jnp.minimum(m1, v)
    m1 = jnp.maximum(m1, v)
    u = jnp.minimum(m2, t)
    m2 = jnp.maximum(m2, t)
    w = jnp.minimum(m3, u)
    m3 = jnp.maximum(m3, u)
    m4 = jnp.maximum(m4, w)
    return m1, m2, m3, m4


def _sc_body(x_hbm, out_hbm, in0, in1, oacc, si0, si1):
    wid = lax.axis_index("s") * NC + lax.axis_index("c")
    wbase = wid * PER_W

    bufs = (in0, in1)
    sems = (si0, si1)

    def issue(c, b):
        base = wbase + c * CH
        for p in range(NPLANE):
            pltpu.async_copy(
                x_hbm.at[pl.ds(p * PSIZE + base, CH)],
                bufs[b].at[pl.ds(p * CH, CH)],
                sems[b],
            )

    def drain(b):
        pltpu.make_async_copy(
            x_hbm.at[pl.ds(0, NPLANE * CH)], bufs[b], sems[b]
        ).wait()

    def compute(c, b):
        buf = bufs[b]

        def group(g, carry):
            s = g * L

            def load(p):
                return buf[pl.ds(p * CH + s, L)]

            def t4(p0):
                return _sort4(load(p0), load(p0 + 1), load(p0 + 2),
                              load(p0 + 3))

            def t16(p0):
                return _merge44(_merge44(t4(p0), t4(p0 + 4)),
                                _merge44(t4(p0 + 8), t4(p0 + 12)))

            t = _merge44(_merge44(t16(0), t16(16)), t16(32))
            t = _insert4(t, load(48))
            o = ((g >> 3) << 11) + (c << 7) + ((g & 7) << 4)
            oacc[pl.ds(o, L)] = (t[0] + t[1] + t[2] + t[3]) * 0.25
            return carry

        lax.fori_loop(0, GROUPS, group, 0)

    issue(0, 0)

    def pair(i, carry):
        c0 = 2 * i
        issue(c0 + 1, 1)
        drain(0)
        compute(c0, 0)

        @pl.when(i < NCHUNK // 2 - 1)
        def _():
            issue(c0 + 2, 0)

        drain(1)
        compute(c0 + 1, 1)
        return carry

    lax.fori_loop(0, NCHUNK // 2, pair, 0)
    pltpu.sync_copy(oacc, out_hbm.at[pl.ds(wbase, PER_W)])


@jax.jit
def _apool_sc(xf):
    mesh = plsc.VectorSubcoreMesh(
        core_axis_name="c", subcore_axis_name="s", num_cores=NC, num_subcores=NS
    )
    return pl.kernel(
        _sc_body,
        out_type=jax.ShapeDtypeStruct((PSIZE,), jnp.float32),
        mesh=mesh,
        compiler_params=pltpu.CompilerParams(needs_layout_passes=False),
        scratch_types=[
            pltpu.VMEM((NPLANE * CH,), jnp.float32),
            pltpu.VMEM((NPLANE * CH,), jnp.float32),
            pltpu.VMEM((PER_W,), jnp.float32),
            pltpu.SemaphoreType.DMA,
            pltpu.SemaphoreType.DMA,
        ],
    )(xf)


def kernel(input, dim):
    xt = jnp.transpose(input, (2, 3, 0, 1))
    x5 = xt.reshape(49, 32, 8, 16, 128)
    x5 = jnp.transpose(x5, (0, 1, 3, 2, 4))
    xf = x5.reshape(-1)
    out = _apool_sc(xf)
    return out.reshape(256, 2048, 1, 1)
